# baseline (speedup 1.0000x reference)
import jax
import jax.numpy as jnp
from jax import lax
from jax.experimental import pallas as pl
from jax.experimental.pallas import tpu as pltpu
from jax.experimental.pallas import tpu_sc as plsc

_NUM_CORES = 2
_NUM_SUBCORES = 16
_NUM_WORKERS = _NUM_CORES * _NUM_SUBCORES
_LANES = 16


def _make_body(rows_per_worker, seq_len, n_out):
    groups = rows_per_worker // _LANES

    chunk_cols = 128
    n_chunks = rows_per_worker // chunk_cols
    groups_per_chunk = chunk_cols // _LANES

    def body(idsT_hbm, coef_hbm, out_hbm, buf0, buf1, coefv, outv, sem0, sem1):
        cid = lax.axis_index("c")
        sid = lax.axis_index("s")
        wid = sid * _NUM_CORES + cid
        base = wid * rows_per_worker
        pltpu.sync_copy(coef_hbm, coefv)
        lane = lax.iota(jnp.int32, 16)

        byte_mask = jnp.full((16,), 0x01010101, jnp.int32)
        byte_sum = jnp.full((16,), 0x01010101, jnp.int32)
        bufs = (buf0, buf1)
        sems = (sem0, sem1)

        def copy_chunk(ch):
            return pltpu.make_async_copy(
                idsT_hbm.at[:, pl.ds(base + ch * chunk_cols, chunk_cols)],
                bufs[ch % 2],
                sems[ch % 2],
            )

        copy_chunk(0).start()
        for ch in range(n_chunks):
            if ch + 1 < n_chunks:
                copy_chunk(ch + 1).start()
            copy_chunk(ch).wait()
            buf = bufs[ch % 2]

            def group(g, _, buf=buf, ch=ch):
                bb0 = g * _LANES
                zero = jnp.zeros((16,), jnp.int32)
                s0 = s1 = s01 = zero
                for st in range(seq_len // 4):
                    l = 4 * st
                    x0 = buf[l, pl.ds(bb0, 16)]
                    x1 = buf[l + 1, pl.ds(bb0, 16)]
                    x2 = buf[l + 2, pl.ds(bb0, 16)]
                    x3 = buf[l + 3, pl.ds(bb0, 16)]
                    c = x0 | (x1 << 8) | (x2 << 16) | (x3 << 24)
                    t0 = c & byte_mask
                    t1 = (c >> 1) & byte_mask
                    s0 = s0 + t0
                    s1 = s1 + t1
                    s01 = s01 + (t0 & t1)

                def byte_total(v):
                    return lax.shift_right_logical(v * byte_sum, 24).astype(jnp.float32)

                f0 = byte_total(s0)
                f1 = byte_total(s1)
                f01 = byte_total(s01)
                bb = ch * chunk_cols + g * _LANES + lane
                for j in range(n_out):
                    v = coefv[pl.ds((4 * j) * 16, 16)] + coefv[pl.ds((4 * j + 1) * 16, 16)] * f0
                    v = v + coefv[pl.ds((4 * j + 2) * 16, 16)] * f1 + coefv[pl.ds((4 * j + 3) * 16, 16)] * f01
                    plsc.store_scatter(outv, [jnp.full((16,), j, jnp.int32), bb], v)
                return 0

            lax.fori_loop(0, groups_per_chunk, group, 0)

        pltpu.sync_copy(outv, out_hbm.at[:, pl.ds(base, rows_per_worker)])

    return body


def kernel(input_ids, embedding_table, dense_w):
    batch, seq_len = input_ids.shape
    n_out = dense_w.shape[1]
    assert batch % (_NUM_WORKERS * _LANES) == 0
    assert seq_len % 8 == 0
    rows_per_worker = batch // _NUM_WORKERS

    m = embedding_table.astype(jnp.float32) @ dense_w.astype(jnp.float32)
    a = m[0]
    b = m[1] - m[0]
    c = m[2] - m[0]
    d = m[3] - m[1] - m[2] + m[0]
    k = seq_len * a
    coef = jnp.stack([k, b, c, d], axis=0).T.reshape(4 * n_out)
    coef = jnp.broadcast_to(coef[:, None], (4 * n_out, _LANES)).reshape(-1)

    ids_t = input_ids.astype(jnp.int32).T

    fn = pl.kernel(
        _make_body(rows_per_worker, seq_len, n_out),
        out_type=jax.ShapeDtypeStruct((n_out, batch), jnp.float32),
        mesh=plsc.VectorSubcoreMesh(
            core_axis_name="c",
            subcore_axis_name="s",
            num_cores=_NUM_CORES,
            num_subcores=_NUM_SUBCORES,
        ),
        scratch_types=[
            pltpu.VMEM((seq_len, 128), jnp.int32),
            pltpu.VMEM((seq_len, 128), jnp.int32),
            pltpu.VMEM((4 * n_out * _LANES,), jnp.float32),
            pltpu.VMEM((n_out, rows_per_worker), jnp.float32),
            pltpu.SemaphoreType.DMA,
            pltpu.SemaphoreType.DMA,
        ],
        compiler_params=pltpu.CompilerParams(
            use_tc_tiling_on_sc=True, needs_layout_passes=False
        ),
    )
    return fn(ids_t, coef).T


# speedup vs baseline: 1.0491x; 1.0491x over previous
import jax
import jax.numpy as jnp
from jax import lax
from jax.experimental import pallas as pl
from jax.experimental.pallas import tpu as pltpu
from jax.experimental.pallas import tpu_sc as plsc

_NUM_CORES = 2
_NUM_SUBCORES = 16
_NUM_WORKERS = _NUM_CORES * _NUM_SUBCORES
_LANES = 16


def _make_body(rows_per_worker, seq_len, n_out):
    groups = rows_per_worker // _LANES

    def body(idsT_hbm, coef_hbm, out_hbm, buf, coefv, outv):
        cid = lax.axis_index("c")
        sid = lax.axis_index("s")
        wid = sid * _NUM_CORES + cid
        base = wid * rows_per_worker
        pltpu.sync_copy(idsT_hbm.at[:, pl.ds(base, rows_per_worker)], buf)
        pltpu.sync_copy(coef_hbm, coefv)
        lane = lax.iota(jnp.int32, 16)

        byte_mask = jnp.full((16,), 0x01010101, jnp.int32)
        byte_sum = jnp.full((16,), 0x01010101, jnp.int32)

        def group(g, _):
            bb0 = g * _LANES
            zero = jnp.zeros((16,), jnp.int32)
            s0 = s1 = s01 = zero
            for st in range(seq_len // 4):
                l = 4 * st
                x0 = buf[l, pl.ds(bb0, 16)]
                x1 = buf[l + 1, pl.ds(bb0, 16)]
                x2 = buf[l + 2, pl.ds(bb0, 16)]
                x3 = buf[l + 3, pl.ds(bb0, 16)]
                p01 = plsc.pack(x0, x1, format=plsc.PackFormat.INTERLEAVED)
                p23 = plsc.pack(x2, x3, format=plsc.PackFormat.INTERLEAVED)
                c8 = plsc.pack(
                    p01, p23,
                    format=plsc.PackFormat.INTERLEAVED,
                    preferred_element_type=jnp.int8,
                )
                c = plsc.bitcast(c8, jnp.int32)
                t0 = c & byte_mask
                t1 = (c >> 1) & byte_mask
                s0 = s0 + t0
                s1 = s1 + t1
                s01 = s01 + (t0 & t1)

            def byte_total(v):
                return lax.shift_right_logical(v * byte_sum, 24).astype(jnp.float32)

            f0 = byte_total(s0)
            f1 = byte_total(s1)
            f01 = byte_total(s01)
            bb = g * _LANES + lane
            for j in range(n_out):
                v = coefv[pl.ds((4 * j) * 16, 16)] + coefv[pl.ds((4 * j + 1) * 16, 16)] * f0
                v = v + coefv[pl.ds((4 * j + 2) * 16, 16)] * f1 + coefv[pl.ds((4 * j + 3) * 16, 16)] * f01
                plsc.store_scatter(outv, [jnp.full((16,), j, jnp.int32), bb], v)
            return 0

        lax.fori_loop(0, groups, group, 0)
        pltpu.sync_copy(outv, out_hbm.at[:, pl.ds(base, rows_per_worker)])

    return body


def kernel(input_ids, embedding_table, dense_w):
    batch, seq_len = input_ids.shape
    n_out = dense_w.shape[1]
    assert batch % (_NUM_WORKERS * _LANES) == 0
    assert seq_len % 8 == 0
    rows_per_worker = batch // _NUM_WORKERS

    m = embedding_table.astype(jnp.float32) @ dense_w.astype(jnp.float32)
    a = m[0]
    b = m[1] - m[0]
    c = m[2] - m[0]
    d = m[3] - m[1] - m[2] + m[0]
    k = seq_len * a
    coef = jnp.stack([k, b, c, d], axis=0).T.reshape(4 * n_out)
    coef = jnp.broadcast_to(coef[:, None], (4 * n_out, _LANES)).reshape(-1)

    ids_t = input_ids.astype(jnp.int32).T

    fn = pl.kernel(
        _make_body(rows_per_worker, seq_len, n_out),
        out_type=jax.ShapeDtypeStruct((n_out, batch), jnp.float32),
        mesh=plsc.VectorSubcoreMesh(
            core_axis_name="c",
            subcore_axis_name="s",
            num_cores=_NUM_CORES,
            num_subcores=_NUM_SUBCORES,
        ),
        scratch_types=[
            pltpu.VMEM((seq_len, rows_per_worker), jnp.int32),
            pltpu.VMEM((4 * n_out * _LANES,), jnp.float32),
            pltpu.VMEM((n_out, rows_per_worker), jnp.float32),
        ],
        compiler_params=pltpu.CompilerParams(
            use_tc_tiling_on_sc=True, needs_layout_passes=False
        ),
    )
    return fn(ids_t, coef).T
